# baseline (speedup 1.0000x reference)
import functools

import jax
import jax.numpy as jnp
from jax import lax
from jax.experimental import pallas as pl
from jax.experimental.pallas import tpu as pltpu
from jax.experimental.pallas import tpu_sc as plsc

BATCH = 16384
HIST = 200
D_FIELD = 16

BPW = 512
KT = 2
KR = 8
BT = 4


@functools.cache
def _build(n_batch, n_vocab):
    info = plsc.get_sparse_core_info()
    nc = info.num_cores

    mesh = plsc.VectorSubcoreMesh(core_axis_name="c", subcore_axis_name="s")

    @functools.partial(
        pl.kernel,
        mesh=mesh,
        out_type=jax.ShapeDtypeStruct((HIST, KT, 128, KR, 128), jnp.float32),
        scratch_types=[
            pltpu.VMEM((2, BPW), jnp.int32),
            pltpu.VMEM((2, BPW, D_FIELD), jnp.float32),
            pltpu.VMEM((2, KT, BT, KR, 128), jnp.float32),
            pltpu.SemaphoreType.DMA((2,)),
            pltpu.SemaphoreType.DMA((2,)),
            pltpu.SemaphoreType.DMA((2,)),
        ],
        compiler_params=pltpu.CompilerParams(
            use_tc_tiling_on_sc=False, needs_layout_passes=False
        ),
    )
    def gather_kernel(idx_hbm, table_hbm, out_hbm, idx_v, rows_v, stg_v, sem_i, sem_g, sem_o):
        wid = lax.axis_index("s") * nc + lax.axis_index("c")
        lane = lax.iota(jnp.int32, 16)

        def idx_off(h, tcl):
            return (
                lax.div(h, 8) * 131072
                + (BT * wid + tcl) * 1024
                + lax.rem(h, 8) * 128
            )

        def fetch_idx(h, slot):
            for tcl in range(BT):
                pltpu.async_copy(
                    idx_hbm.at[pl.ds(idx_off(h, tcl), 128)],
                    idx_v.at[slot, pl.ds(tcl * 128, 128)],
                    sem_i.at[slot],
                )

        fetch_idx(0, 0)
        fetch_idx(1, 1)

        def step(p, carry):
            for s in range(2):
                i = 2 * p + s
                sj = 1 - s

                @pl.when(i < HIST)
                def _fire():
                    for tcl in range(BT):
                        pltpu.make_async_copy(
                            idx_hbm.at[pl.ds(0, 128)],
                            idx_v.at[s, pl.ds(tcl * 128, 128)],
                            sem_i.at[s],
                        ).wait()
                    for tcl in range(BT):
                        pltpu.async_copy(
                            table_hbm.at[idx_v.at[s, pl.ds(tcl * 128, 128)]],
                            rows_v.at[s, pl.ds(tcl * 128, 128)],
                            sem_g.at[s],
                        )

                @pl.when((i >= 1) & (i <= HIST))
                def _finish():
                    j = i - 1
                    for tcl in range(BT):
                        pltpu.make_async_copy(
                            table_hbm.at[idx_v.at[sj, pl.ds(tcl * 128, 128)]],
                            rows_v.at[sj, pl.ds(tcl * 128, 128)],
                            sem_g.at[sj],
                        ).wait()

                    @pl.when(i + 1 < HIST)
                    def _prefetch_idx():
                        fetch_idx(i + 1, sj)

                    @pl.when(j >= 2)
                    def _wait_out():
                        pltpu.make_async_copy(
                            stg_v.at[sj],
                            out_hbm.at[0, :, pl.ds(wid * BT, BT)],
                            sem_o.at[sj],
                        ).wait()

                    rows2d = rows_v.at[sj]
                    for kt in range(KT):
                        for bt in range(BT):
                            for kr in range(KR):
                                col = jnp.full((16,), kt * KR + kr, jnp.int32)
                                vs = [
                                    plsc.load_gather(
                                        rows2d,
                                        [lane + (bt * 128 + cb * 16), col],
                                    )
                                    for cb in range(8)
                                ]
                                for cb in range(8):
                                    stg_v[sj, kt, bt, kr, pl.ds(cb * 16, 16)] = (
                                        vs[cb]
                                    )

                    pltpu.async_copy(
                        stg_v.at[sj],
                        out_hbm.at[j, :, pl.ds(wid * BT, BT)],
                        sem_o.at[sj],
                    )

            return carry

        lax.fori_loop(0, HIST // 2 + 1, step, 0)

        for s in range(2):
            pltpu.make_async_copy(
                stg_v.at[s], out_hbm.at[0, :, pl.ds(wid * BT, BT)], sem_o.at[s]
            ).wait()

    return gather_kernel


def kernel(lookup, table):
    idx_flat = (
        lookup.T.astype(jnp.int32)
        .reshape(25, 8, 128, 128)
        .transpose(0, 2, 1, 3)
        .reshape(BATCH * HIST)
    )
    t5 = _build(lookup.shape[0], table.shape[0])(idx_flat, table)
    return (
        t5.transpose(0, 1, 3, 2, 4)
        .reshape(HIST, D_FIELD, BATCH)
        .transpose(2, 0, 1)
    )


# speedup vs baseline: 1.0001x; 1.0001x over previous
import functools

import jax
import jax.numpy as jnp
from jax import lax
from jax.experimental import pallas as pl
from jax.experimental.pallas import tpu as pltpu
from jax.experimental.pallas import tpu_sc as plsc

BATCH = 16384
HIST = 200
D_FIELD = 16

BPW = 512
KT = 2
KR = 8
BT = 4
NS = 4


@functools.cache
def _build(n_batch, n_vocab):
    info = plsc.get_sparse_core_info()
    nc = info.num_cores

    mesh = plsc.VectorSubcoreMesh(core_axis_name="c", subcore_axis_name="s")

    @functools.partial(
        pl.kernel,
        mesh=mesh,
        out_type=jax.ShapeDtypeStruct((HIST, KT, 128, KR, 128), jnp.float32),
        scratch_types=[
            pltpu.VMEM((NS, BPW), jnp.int32),
            pltpu.VMEM((NS, BPW, D_FIELD), jnp.float32),
            pltpu.VMEM((NS, KT, BT, KR, 128), jnp.float32),
            pltpu.SemaphoreType.DMA((NS,)),
            pltpu.SemaphoreType.DMA((NS,)),
            pltpu.SemaphoreType.DMA((NS,)),
        ],
        compiler_params=pltpu.CompilerParams(
            use_tc_tiling_on_sc=False, needs_layout_passes=False
        ),
    )
    def gather_kernel(idx_hbm, table_hbm, out_hbm, idx_v, rows_v, stg_v, sem_i, sem_g, sem_o):
        wid = lax.axis_index("s") * nc + lax.axis_index("c")
        lane = lax.iota(jnp.int32, 16)

        def fetch_idx(h, slot):
            for tcl in range(BT):
                off = (
                    lax.div(h, 8) * 131072
                    + (BT * wid + tcl) * 1024
                    + lax.rem(h, 8) * 128
                )
                pltpu.async_copy(
                    idx_hbm.at[pl.ds(off, 128)],
                    idx_v.at[slot, pl.ds(tcl * 128, 128)],
                    sem_i.at[slot],
                )

        def wait_idx(slot):
            pltpu.make_async_copy(
                idx_hbm.at[pl.ds(0, BPW)], idx_v.at[slot], sem_i.at[slot]
            ).wait()

        for s in range(NS):
            fetch_idx(s, s)

        def step(p, carry):
            for s in range(NS):
                i = NS * p + s

                @pl.when(i < HIST)
                def _fire():
                    wait_idx(s)
                    pltpu.async_copy(
                        table_hbm.at[idx_v.at[s]], rows_v.at[s], sem_g.at[s]
                    )

                @pl.when((i >= 2) & (i < HIST + 2))
                def _finish():
                    j = i - 2
                    sj = (s - 2) % NS
                    pltpu.make_async_copy(
                        table_hbm.at[idx_v.at[sj]], rows_v.at[sj], sem_g.at[sj]
                    ).wait()

                    @pl.when(j + NS < HIST)
                    def _prefetch_idx():
                        fetch_idx(j + NS, sj)

                    @pl.when(j >= NS)
                    def _wait_out():
                        pltpu.make_async_copy(
                            stg_v.at[sj],
                            out_hbm.at[0, :, pl.ds(wid * BT, BT)],
                            sem_o.at[sj],
                        ).wait()

                    rows2d = rows_v.at[sj]
                    for kt in range(KT):
                        for bt in range(BT):
                            for kr in range(KR):
                                col = jnp.full((16,), kt * KR + kr, jnp.int32)
                                vs = [
                                    plsc.load_gather(
                                        rows2d,
                                        [lane + (bt * 128 + cb * 16), col],
                                    )
                                    for cb in range(8)
                                ]
                                for cb in range(8):
                                    stg_v[sj, kt, bt, kr, pl.ds(cb * 16, 16)] = (
                                        vs[cb]
                                    )

                    pltpu.async_copy(
                        stg_v.at[sj],
                        out_hbm.at[j, :, pl.ds(wid * BT, BT)],
                        sem_o.at[sj],
                    )

            return carry

        lax.fori_loop(0, (HIST + 2 + NS - 1) // NS + 1, step, 0)

        for s in range(NS):
            pltpu.make_async_copy(
                stg_v.at[s], out_hbm.at[0, :, pl.ds(wid * BT, BT)], sem_o.at[s]
            ).wait()

    return gather_kernel


def kernel(lookup, table):
    idx_flat = (
        lookup.T.astype(jnp.int32)
        .reshape(25, 8, 128, 128)
        .transpose(0, 2, 1, 3)
        .reshape(BATCH * HIST)
    )
    t5 = _build(lookup.shape[0], table.shape[0])(idx_flat, table)
    return (
        t5.transpose(0, 1, 3, 2, 4)
        .reshape(HIST, D_FIELD, BATCH)
        .transpose(2, 0, 1)
    )


# speedup vs baseline: 1.0131x; 1.0131x over previous
import functools

import jax
import jax.numpy as jnp
from jax import lax
from jax.experimental import pallas as pl
from jax.experimental.pallas import tpu as pltpu
from jax.experimental.pallas import tpu_sc as plsc

BATCH = 16384
HIST = 200
D_FIELD = 16

BPW = 512
KT = 2
BT = 4
HC = 4
NCHUNK = HIST // HC
CROWS = HC * BPW
HPLANE = KT * 128 * 8 * 128
WBLK = 8 * 128 * BT


@functools.cache
def _build(n_batch, n_vocab):
    info = plsc.get_sparse_core_info()
    nc = info.num_cores

    mesh = plsc.VectorSubcoreMesh(core_axis_name="c", subcore_axis_name="s")

    @functools.partial(
        pl.kernel,
        mesh=mesh,
        out_type=jax.ShapeDtypeStruct((BATCH * HIST * D_FIELD,), jnp.float32),
        scratch_types=[
            pltpu.VMEM((2, CROWS), jnp.int32),
            pltpu.VMEM((2, CROWS, D_FIELD), jnp.float32),
            pltpu.VMEM((HC * KT * WBLK,), jnp.float32),
            pltpu.SemaphoreType.DMA((2,)),
            pltpu.SemaphoreType.DMA((2,)),
            pltpu.SemaphoreType.DMA((HC,)),
        ],
        compiler_params=pltpu.CompilerParams(
            use_tc_tiling_on_sc=False, needs_layout_passes=False
        ),
    )
    def gather_kernel(idx_hbm, table_hbm, out_hbm, idx_v, rows_v, stg_v, sem_i, sem_g, sem_o):
        wid = lax.axis_index("s") * nc + lax.axis_index("c")
        lane = lax.iota(jnp.int32, 16)

        def fetch_idx(tr, r0, slot):
            for tcl in range(BT):
                off = tr * 131072 + (BT * wid + tcl) * 1024 + r0 * 128
                pltpu.async_copy(
                    idx_hbm.at[pl.ds(off, HC * 128)],
                    idx_v.at[slot, pl.ds(tcl * HC * 128, HC * 128)],
                    sem_i.at[slot],
                )

        def wait_idx(slot):
            pltpu.make_async_copy(
                idx_hbm.at[pl.ds(0, CROWS)], idx_v.at[slot], sem_i.at[slot]
            ).wait()

        fetch_idx(0, 0, 0)
        fetch_idx(0, 4, 1)

        def transpose_h(j, hl, sr):
            rows2d = rows_v.at[sr]

            @pl.when(j >= 1)
            def _reclaim():
                pltpu.make_async_copy(
                    stg_v.at[pl.ds(0, KT * WBLK)],
                    out_hbm.at[pl.ds(0, KT * WBLK)],
                    sem_o.at[hl],
                ).wait()
            for kt in range(KT):
                for tcl in range(BT):
                    for kr in range(8):
                        col = jnp.full((16,), kt * 8 + kr, jnp.int32)
                        vs = [
                            plsc.load_gather(
                                rows2d,
                                [
                                    lane
                                    + (hl * 128 + (tcl * HC * 128 + cb * 16)),
                                    col,
                                ],
                            )
                            for cb in range(8)
                        ]
                        base = hl * KT * WBLK + kt * WBLK + tcl * 1024 + kr * 128
                        for cb in range(8):
                            stg_v[pl.ds(base + cb * 16, 16)] = vs[cb]
            h = HC * j + hl
            for kt in range(KT):
                pltpu.async_copy(
                    stg_v.at[pl.ds(hl * KT * WBLK + kt * WBLK, WBLK)],
                    out_hbm.at[
                        pl.ds(h * HPLANE + kt * (HPLANE // KT) + wid * WBLK, WBLK)
                    ],
                    sem_o.at[hl],
                )

        def step(p, carry):
            for s in range(2):
                g = 2 * p + s

                @pl.when(g < NCHUNK)
                def _fire():
                    wait_idx(s)
                    pltpu.async_copy(
                        table_hbm.at[idx_v.at[s]], rows_v.at[s], sem_g.at[s]
                    )

                @pl.when((g >= 1) & (g <= NCHUNK))
                def _finish():
                    j = g - 1
                    sr = 1 - s
                    pltpu.make_async_copy(
                        table_hbm.at[idx_v.at[sr]], rows_v.at[sr], sem_g.at[sr]
                    ).wait()

                    @pl.when(j + 2 < NCHUNK)
                    def _prefetch_idx():
                        fetch_idx(p + s, 4 * sr, sr)

                    def hl_body(hl, c2):
                        transpose_h(j, hl, sr)
                        return c2

                    lax.fori_loop(0, HC, hl_body, 0)

            return carry

        lax.fori_loop(0, NCHUNK // 2 + 1, step, 0)

        for hl in range(HC):
            for kt in range(KT):
                pltpu.make_async_copy(
                    stg_v.at[pl.ds(0, WBLK)],
                    out_hbm.at[pl.ds(0, WBLK)],
                    sem_o.at[hl],
                ).wait()

    return gather_kernel


def kernel(lookup, table):
    idx_flat = (
        lookup.T.astype(jnp.int32)
        .reshape(25, 8, 128, 128)
        .transpose(0, 2, 1, 3)
        .reshape(BATCH * HIST)
    )
    flat = _build(lookup.shape[0], table.shape[0])(idx_flat, table)
    return (
        flat.reshape(HIST, KT, 128, 8, 128)
        .transpose(0, 1, 3, 2, 4)
        .reshape(HIST, D_FIELD, BATCH)
        .transpose(2, 0, 1)
    )
